# baseline (speedup 1.0000x reference)
import functools

import jax
import jax.numpy as jnp
from jax import lax
from jax.experimental import pallas as pl
from jax.experimental.pallas import tpu as pltpu
from jax.experimental.pallas import tpu_sc as plsc

B, N, D = 1024, 50, 512
NUM_TYPES = 4
NC, NS, L = 2, 16, 16
NW = NC * NS
BPW = B // NW
N_PAD = 56


def _make_kernel():
    mesh = plsc.VectorSubcoreMesh(core_axis_name="c", subcore_axis_name="s")

    @functools.partial(
        pl.kernel,
        mesh=mesh,
        out_type=jax.ShapeDtypeStruct((B, N, D), jnp.float32),
        scratch_types=[
            pltpu.VMEM((N_PAD,), jnp.int32),
            pltpu.VMEM((N_PAD, D), jnp.float32),
            pltpu.VMEM((N, D), jnp.float32),
            pltpu.SemaphoreType.DMA,
        ],
    )
    def k(cs_hbm, tid_hbm, emb_hbm, out_hbm, tid_v, temb_v, buf, sem):
        wid = lax.axis_index("s") * NC + lax.axis_index("c")
        base = wid * BPW

        pltpu.sync_copy(tid_hbm, tid_v)
        pltpu.async_copy(emb_hbm.at[tid_v], temb_v, sem).wait()

        def batch_body(t, carry):
            b = base + t
            pltpu.sync_copy(cs_hbm.at[b], buf)

            def row_body(i, c):
                for j in range(D // L):
                    sl = pl.ds(j * L, L)
                    buf[i, sl] = buf[i, sl] + temb_v[i, sl]
                return c

            lax.fori_loop(0, N, row_body, 0)
            pltpu.sync_copy(buf, out_hbm.at[b])
            return carry

        lax.fori_loop(0, BPW, batch_body, 0)

    return k


_k = _make_kernel()


def kernel(channel_stack, type_ids, embeddings):
    tid = jnp.zeros((N_PAD,), jnp.int32).at[:N].set(type_ids.astype(jnp.int32))
    return _k(channel_stack, tid, embeddings)
